# Initial kernel scaffold; baseline (speedup 1.0000x reference)
import jax
import jax.numpy as jnp
from jax.experimental import pallas as pl


def kernel(x, ln1_g, ln1_b, w1, b1, ln2_g, ln2_b, w2, b2):
    raise NotImplementedError("write your pallas kernel here")


# speedup vs baseline: 2.9511x; 2.9511x over previous
import math

import jax
import jax.numpy as jnp
from jax.experimental import pallas as pl
from jax.experimental.pallas import tpu as pltpu

_LN_EPS = 1e-5
_BM = 256


def _wquant_body(w_ref, t_ref):
    w = w_ref[...]
    a = jnp.abs(w)
    m = jnp.sum(a, axis=0, keepdims=True)
    m = jnp.sum(m, axis=1, keepdims=True) * (1.0 / (w.shape[0] * w.shape[1]))
    m = jnp.maximum(m, 1e-5)
    t = jnp.clip(jnp.round(w * (1.0 / m)), -1.0, 1.0)
    t_ref[...] = (t * m).astype(jnp.bfloat16)


def _quantize_weight(w):
    return pl.pallas_call(
        _wquant_body,
        out_shape=jax.ShapeDtypeStruct(w.shape, jnp.bfloat16),
        compiler_params=pltpu.CompilerParams(
            vmem_limit_bytes=48 * 1024 * 1024),
    )(w)


def _layernorm(x, g, b):
    mu = jnp.mean(x, axis=-1, keepdims=True)
    xc = x - mu
    var = jnp.mean(xc * xc, axis=-1, keepdims=True)
    return xc * jax.lax.rsqrt(var + _LN_EPS) * g + b


def _act_quant(h):
    amax = jnp.maximum(jnp.max(jnp.abs(h), axis=-1, keepdims=True), 1e-5)
    q = jnp.clip(jnp.round(h * (127.0 / amax)), -128.0, 127.0)
    return (q * (amax * (1.0 / 127.0))).astype(jnp.bfloat16)


def _ffn_body(x_ref, g1_ref, bl1_ref, w1t_ref, b1_ref,
              g2_ref, bl2_ref, w2t_ref, b2_ref, o_ref):
    h = _layernorm(x_ref[...], g1_ref[...], bl1_ref[...])
    dqx1 = _act_quant(h)
    u = jax.lax.dot_general(
        dqx1, w1t_ref[...], (((1,), (0,)), ((), ())),
        preferred_element_type=jnp.float32) + b1_ref[...]

    g = 0.5 * u * (1.0 + jax.lax.erf(u * (1.0 / math.sqrt(2.0))))

    h2 = _layernorm(g, g2_ref[...], bl2_ref[...])
    dqx2 = _act_quant(h2)
    o_ref[...] = jax.lax.dot_general(
        dqx2, w2t_ref[...], (((1,), (0,)), ((), ())),
        preferred_element_type=jnp.float32) + b2_ref[...]


def kernel(x, ln1_g, ln1_b, w1, b1, ln2_g, ln2_b, w2, b2):
    B, T, D = x.shape
    H = w1.shape[0]
    M = B * T
    x2 = x.reshape(M, D)

    w1t = _quantize_weight(w1).T
    w2t = _quantize_weight(w2).T

    full = lambda shape: pl.BlockSpec(shape, lambda i: (0, 0))
    out = pl.pallas_call(
        _ffn_body,
        grid=(M // _BM,),
        in_specs=[
            pl.BlockSpec((_BM, D), lambda i: (i, 0)),
            full((1, D)), full((1, D)),
            full((D, H)), full((1, H)),
            full((1, H)), full((1, H)),
            full((H, D)), full((1, D)),
        ],
        out_specs=pl.BlockSpec((_BM, D), lambda i: (i, 0)),
        out_shape=jax.ShapeDtypeStruct((M, D), jnp.float32),
        compiler_params=pltpu.CompilerParams(
            dimension_semantics=("parallel",),
            vmem_limit_bytes=56 * 1024 * 1024),
    )(x2, ln1_g.reshape(1, D), ln1_b.reshape(1, D), w1t, b1.reshape(1, H),
      ln2_g.reshape(1, H), ln2_b.reshape(1, H), w2t, b2.reshape(1, D))
    return out.reshape(B, T, D)


# speedup vs baseline: 3.1092x; 1.0536x over previous
import math

import jax
import jax.numpy as jnp
from jax.experimental import pallas as pl
from jax.experimental.pallas import tpu as pltpu

_LN_EPS = 1e-5
_BM = 512
_CHAINS = 2


def _wquant_body(w_ref, t_ref):
    w = w_ref[...]
    a = jnp.abs(w)
    m = jnp.sum(a, axis=0, keepdims=True)
    m = jnp.sum(m, axis=1, keepdims=True) * (1.0 / (w.shape[0] * w.shape[1]))
    m = jnp.maximum(m, 1e-5)
    t = jnp.clip(jnp.round(w * (1.0 / m)), -1.0, 1.0)
    t_ref[...] = (t * m).astype(jnp.bfloat16)


def _quantize_weight(w):
    return pl.pallas_call(
        _wquant_body,
        out_shape=jax.ShapeDtypeStruct(w.shape, jnp.bfloat16),
        compiler_params=pltpu.CompilerParams(
            vmem_limit_bytes=48 * 1024 * 1024),
    )(w)


def _layernorm(x, g, b):
    mu = jnp.mean(x, axis=-1, keepdims=True)
    xc = x - mu
    var = jnp.mean(xc * xc, axis=-1, keepdims=True)
    return xc * jax.lax.rsqrt(var + _LN_EPS) * g + b


def _act_quant(h):
    amax = jnp.maximum(jnp.max(jnp.abs(h), axis=-1, keepdims=True), 1e-5)
    q = jnp.clip(jnp.round(h * (127.0 / amax)), -128.0, 127.0)
    return (q * (amax * (1.0 / 127.0))).astype(jnp.bfloat16)


def _ffn_body(x_ref, g1_ref, bl1_ref, w1t_ref, b1_ref,
              g2_ref, bl2_ref, w2t_ref, b2_ref, o_ref):
    def chain(rows):
        h = _layernorm(x_ref[rows, :], g1_ref[...], bl1_ref[...])
        dqx1 = _act_quant(h)
        u = jax.lax.dot_general(
            dqx1, w1t_ref[...], (((1,), (0,)), ((), ())),
            preferred_element_type=jnp.float32) + b1_ref[...]

        g = 0.5 * u * (1.0 + jax.lax.erf(u * (1.0 / math.sqrt(2.0))))

        h2 = _layernorm(g, g2_ref[...], bl2_ref[...])
        dqx2 = _act_quant(h2)
        o_ref[rows, :] = jax.lax.dot_general(
            dqx2, w2t_ref[...], (((1,), (0,)), ((), ())),
            preferred_element_type=jnp.float32) + b2_ref[...]

    sub = _BM // _CHAINS
    for c in range(_CHAINS):
        chain(slice(c * sub, (c + 1) * sub))


def kernel(x, ln1_g, ln1_b, w1, b1, ln2_g, ln2_b, w2, b2):
    B, T, D = x.shape
    H = w1.shape[0]
    M = B * T
    x2 = x.reshape(M, D)

    w1t = _quantize_weight(w1).T
    w2t = _quantize_weight(w2).T

    full = lambda shape: pl.BlockSpec(shape, lambda i: (0, 0))
    out = pl.pallas_call(
        _ffn_body,
        grid=(M // _BM,),
        in_specs=[
            pl.BlockSpec((_BM, D), lambda i: (i, 0)),
            full((1, D)), full((1, D)),
            full((D, H)), full((1, H)),
            full((1, H)), full((1, H)),
            full((H, D)), full((1, D)),
        ],
        out_specs=pl.BlockSpec((_BM, D), lambda i: (i, 0)),
        out_shape=jax.ShapeDtypeStruct((M, D), jnp.float32),
        compiler_params=pltpu.CompilerParams(
            dimension_semantics=("parallel",),
            vmem_limit_bytes=56 * 1024 * 1024),
    )(x2, ln1_g.reshape(1, D), ln1_b.reshape(1, D), w1t, b1.reshape(1, H),
      ln2_g.reshape(1, H), ln2_b.reshape(1, H), w2t, b2.reshape(1, D))
    return out.reshape(B, T, D)


# speedup vs baseline: 3.3855x; 1.0889x over previous
import math

import jax
import jax.numpy as jnp
from jax.experimental import pallas as pl
from jax.experimental.pallas import tpu as pltpu

_LN_EPS = 1e-5
_BM = 1024


def _wquant_body(w_ref, t_ref):
    w = w_ref[...]
    a = jnp.abs(w)
    m = jnp.sum(a, axis=0, keepdims=True)
    m = jnp.sum(m, axis=1, keepdims=True) * (1.0 / (w.shape[0] * w.shape[1]))
    m = jnp.maximum(m, 1e-5)
    t = jnp.clip(jnp.round(w * (1.0 / m)), -1.0, 1.0)
    t_ref[...] = (t * m).astype(jnp.bfloat16)


def _quantize_weight(w):
    return pl.pallas_call(
        _wquant_body,
        out_shape=jax.ShapeDtypeStruct(w.shape, jnp.bfloat16),
        compiler_params=pltpu.CompilerParams(
            vmem_limit_bytes=48 * 1024 * 1024),
    )(w)


def _layernorm(x, g, b):
    mu = jnp.mean(x, axis=-1, keepdims=True)
    xc = x - mu
    var = jnp.mean(xc * xc, axis=-1, keepdims=True)
    return xc * jax.lax.rsqrt(var + _LN_EPS) * g + b


def _act_quant(h):
    amax = jnp.maximum(jnp.max(jnp.abs(h), axis=-1, keepdims=True), 1e-5)
    q = jnp.clip(jnp.round(h * (127.0 / amax)), -128.0, 127.0)
    return (q * (amax * (1.0 / 127.0))).astype(jnp.bfloat16)


def _ffn_body(x_ref, g1_ref, bl1_ref, w1t_ref, b1_ref,
              g2_ref, bl2_ref, w2t_ref, b2_ref, o_ref):
    h = _layernorm(x_ref[...], g1_ref[...], bl1_ref[...])
    dqx1 = _act_quant(h)
    u = jax.lax.dot_general(
        dqx1, w1t_ref[...], (((1,), (0,)), ((), ())),
        preferred_element_type=jnp.float32) + b1_ref[...]

    g = 0.5 * u * (1.0 + jax.lax.erf(u * (1.0 / math.sqrt(2.0))))

    h2 = _layernorm(g, g2_ref[...], bl2_ref[...])
    dqx2 = _act_quant(h2)
    o_ref[...] = jax.lax.dot_general(
        dqx2, w2t_ref[...], (((1,), (0,)), ((), ())),
        preferred_element_type=jnp.float32) + b2_ref[...]


def kernel(x, ln1_g, ln1_b, w1, b1, ln2_g, ln2_b, w2, b2):
    B, T, D = x.shape
    H = w1.shape[0]
    M = B * T
    x2 = x.reshape(M, D)

    w1t = _quantize_weight(w1).T
    w2t = _quantize_weight(w2).T

    full = lambda shape: pl.BlockSpec(shape, lambda i: (0, 0))
    out = pl.pallas_call(
        _ffn_body,
        grid=(M // _BM,),
        in_specs=[
            pl.BlockSpec((_BM, D), lambda i: (i, 0)),
            full((1, D)), full((1, D)),
            full((D, H)), full((1, H)),
            full((1, H)), full((1, H)),
            full((H, D)), full((1, D)),
        ],
        out_specs=pl.BlockSpec((_BM, D), lambda i: (i, 0)),
        out_shape=jax.ShapeDtypeStruct((M, D), jnp.float32),
        compiler_params=pltpu.CompilerParams(
            dimension_semantics=("parallel",),
            vmem_limit_bytes=56 * 1024 * 1024),
    )(x2, ln1_g.reshape(1, D), ln1_b.reshape(1, D), w1t, b1.reshape(1, H),
      ln2_g.reshape(1, H), ln2_b.reshape(1, H), w2t, b2.reshape(1, D))
    return out.reshape(B, T, D)
